# baseline (speedup 1.0000x reference)
import jax
import jax.numpy as jnp
from jax.experimental import pallas as pl

N_CLASSES = 81
THRESH = 0.5
NEGPOS = 3
V0, V1 = 0.1, 0.2
N_PRIORS = 8732
N_OBJS = 20
IMGS = 4


def _stage1(tgt_ref, pri_ref, loc_ref, conf_ref, lcn_ref, stats_ref):
    pri = pri_ref[...]
    p_cx = pri[0:1, :]
    p_cy = pri[1:2, :]
    p_w = pri[2:3, :]
    p_h = pri[3:4, :]
    p_x0 = p_cx - p_w * 0.5
    p_y0 = p_cy - p_h * 0.5
    p_x1 = p_cx + p_w * 0.5
    p_y1 = p_cy + p_h * 0.5
    area_p = (p_x1 - p_x0) * (p_y1 - p_y0)

    ti = jax.lax.broadcasted_iota(jnp.int32, (N_OBJS, N_PRIORS), 0)
    ji = jax.lax.broadcasted_iota(jnp.int32, (N_OBJS, N_PRIORS), 1)
    ci = jax.lax.broadcasted_iota(jnp.int32, (N_CLASSES, N_PRIORS), 0)
    ones_cls = jnp.ones((1, N_CLASSES), jnp.float32)
    lane = jax.lax.broadcasted_iota(jnp.int32, (1, 128), 1)

    for i in range(IMGS):
        t = tgt_ref[i]
        tx0 = t[:, 0:1]
        ty0 = t[:, 1:2]
        tx1 = t[:, 2:3]
        ty1 = t[:, 3:4]
        lab = t[:, 4:5]

        iw = jnp.clip(jnp.minimum(tx1, p_x1) - jnp.maximum(tx0, p_x0),
                      0.0, None)
        ih = jnp.clip(jnp.minimum(ty1, p_y1) - jnp.maximum(ty0, p_y0),
                      0.0, None)
        inter = iw * ih
        area_t = (tx1 - tx0) * (ty1 - ty0)
        ov = inter / (area_t + area_p - inter)

        bto = jnp.max(ov, axis=0, keepdims=True)
        bti = jnp.min(jnp.where(ov == bto, ti, N_OBJS), axis=0, keepdims=True)

        bpo = jnp.max(ov, axis=1, keepdims=True)
        bpi = jnp.min(jnp.where(ov == bpo, ji, N_PRIORS), axis=1,
                      keepdims=True)

        eq = ji == bpi
        forced = jnp.max(eq.astype(jnp.int32), axis=0, keepdims=True) > 0
        f_t = jnp.max(jnp.where(eq, ti, -1), axis=0, keepdims=True)
        bto2 = jnp.where(forced, 2.0, bto)
        bti2 = jnp.where(forced, f_t, bti)

        msel = ti == bti2
        mx0 = jnp.sum(jnp.where(msel, tx0, 0.0), axis=0, keepdims=True)
        my0 = jnp.sum(jnp.where(msel, ty0, 0.0), axis=0, keepdims=True)
        mx1 = jnp.sum(jnp.where(msel, tx1, 0.0), axis=0, keepdims=True)
        my1 = jnp.sum(jnp.where(msel, ty1, 0.0), axis=0, keepdims=True)
        mlab = jnp.sum(jnp.where(msel, lab, 0.0), axis=0, keepdims=True)

        pos = bto2 >= THRESH
        posf = pos.astype(jnp.float32)
        confc = jnp.where(pos, mlab + 1.0, 0.0)

        g_cx = ((mx0 + mx1) * 0.5 - p_cx) / (V0 * p_w)
        g_cy = ((my0 + my1) * 0.5 - p_cy) / (V0 * p_h)
        g_w = jnp.log((mx1 - mx0) / p_w) / V1
        g_h = jnp.log((my1 - my0) / p_h) / V1

        l = loc_ref[i]

        def sl1(d):
            a = jnp.abs(d)
            return jnp.where(a < 1.0, 0.5 * d * d, a - 0.5)

        sl = sl1(l[0:1, :] - g_cx) + sl1(l[1:2, :] - g_cy) \
            + sl1(l[2:3, :] - g_w) + sl1(l[3:4, :] - g_h)
        loss_l = jnp.sum(sl * posf)

        c = conf_ref[i].T
        m = jnp.max(c, axis=0, keepdims=True)
        z = jnp.exp(c - m)
        onehot = ci == confc.astype(jnp.int32)
        cm = jnp.where(onehot, c, 0.0)
        s = jax.lax.dot_general(ones_cls, z, (((1,), (0,)), ((), ())),
                                preferred_element_type=jnp.float32)
        xc = jax.lax.dot_general(ones_cls, cm, (((1,), (0,)), ((), ())),
                                 preferred_element_type=jnp.float32)
        lse = jnp.log(s) + m
        loss_c = lse - xc

        pos_loss = jnp.sum(jnp.where(pos, loss_c, 0.0))
        lcn = jnp.where(pos, 0.0, loss_c)
        num_pos = jnp.sum(posf)

        lcn_ref[i] = lcn
        stats_ref[i] = jnp.where(lane == 0, loss_l,
                                 jnp.where(lane == 1, pos_loss,
                                           jnp.where(lane == 2, num_pos,
                                                     0.0)))


def _stage2(lcn_ref, stats_ref, out_ref):
    lcn = lcn_ref[...]
    stats = stats_ref[...]
    loss_l = jnp.sum(stats[:, 0, 0:1])
    pos_loss = jnp.sum(stats[:, 0, 1:2])
    npos = stats[:, 0, 2:3]
    n_total = jnp.sum(npos)
    k = jnp.minimum((npos * float(NEGPOS)).astype(jnp.int32),
                    N_PRIORS - 1)
    bits = jax.lax.bitcast_convert_type(lcn, jnp.int32)

    prefix = jnp.zeros((lcn.shape[0], 1), jnp.int32)
    for b in range(30, -1, -1):
        trial = prefix | (1 << b)
        cnt = jnp.sum((bits >= trial).astype(jnp.int32), axis=1,
                      keepdims=True)
        prefix = jnp.where(cnt >= k, trial, prefix)

    cnt_gt = jnp.sum((bits > prefix).astype(jnp.int32), axis=1, keepdims=True)
    sum_gt = jnp.sum(jnp.where(bits > prefix, lcn, 0.0), axis=1,
                     keepdims=True)
    tval = jax.lax.bitcast_convert_type(prefix, jnp.float32)
    neg = sum_gt + (k - cnt_gt).astype(jnp.float32) * tval
    neg_loss = jnp.sum(jnp.where(k > 0, neg, 0.0))

    lane = jax.lax.broadcasted_iota(jnp.int32, (1, 128), 1)
    out_ref[...] = jnp.where(
        lane == 0, loss_l / n_total,
        jnp.where(lane == 1, (pos_loss + neg_loss) / n_total, 0.0))


def kernel(loc_data, conf_data, targets, priors):
    batch = loc_data.shape[0]
    loc_t = jnp.transpose(loc_data, (0, 2, 1))
    pri_t = priors.T

    lcn, stats = pl.pallas_call(
        _stage1,
        grid=(batch // IMGS,),
        in_specs=[
            pl.BlockSpec((IMGS, N_OBJS, 5), lambda b: (b, 0, 0)),
            pl.BlockSpec((4, N_PRIORS), lambda b: (0, 0)),
            pl.BlockSpec((IMGS, 4, N_PRIORS), lambda b: (b, 0, 0)),
            pl.BlockSpec((IMGS, N_PRIORS, N_CLASSES), lambda b: (b, 0, 0)),
        ],
        out_specs=[
            pl.BlockSpec((IMGS, 1, N_PRIORS), lambda b: (b, 0, 0)),
            pl.BlockSpec((IMGS, 1, 128), lambda b: (b, 0, 0)),
        ],
        out_shape=[
            jax.ShapeDtypeStruct((batch, 1, N_PRIORS), jnp.float32),
            jax.ShapeDtypeStruct((batch, 1, 128), jnp.float32),
        ],
    )(targets, pri_t, loc_t, conf_data)

    out = pl.pallas_call(
        _stage2,
        out_shape=jax.ShapeDtypeStruct((1, 128), jnp.float32),
    )(lcn.reshape(batch, N_PRIORS), stats)

    return (out[0, 0], out[0, 1])


# speedup vs baseline: 1.1226x; 1.1226x over previous
import jax
import jax.numpy as jnp
from jax.experimental import pallas as pl

N_CLASSES = 81
THRESH = 0.5
NEGPOS = 3
V0, V1 = 0.1, 0.2
N_PRIORS = 8732
N_OBJS = 20
IMGS = 4


def _stage1(tgt_ref, pri_ref, loc_ref, conf_ref, lcn_ref, stats_ref):
    pri = pri_ref[...]
    p_cx = pri[0:1, :]
    p_cy = pri[1:2, :]
    p_w = pri[2:3, :]
    p_h = pri[3:4, :]
    p_x0 = p_cx - p_w * 0.5
    p_y0 = p_cy - p_h * 0.5
    p_x1 = p_cx + p_w * 0.5
    p_y1 = p_cy + p_h * 0.5
    area_p = (p_x1 - p_x0) * (p_y1 - p_y0)

    ti = jax.lax.broadcasted_iota(jnp.int32, (N_OBJS, N_PRIORS), 0)
    ji = jax.lax.broadcasted_iota(jnp.int32, (N_OBJS, N_PRIORS), 1)
    ones_cls = jnp.ones((1, N_CLASSES), jnp.float32)
    ones_t = jnp.ones((1, N_OBJS), jnp.float32)
    lane = jax.lax.broadcasted_iota(jnp.int32, (1, 128), 1)
    ti1 = jax.lax.broadcasted_iota(jnp.int32, (N_OBJS, 1), 0)
    pw_col = jax.lax.bitcast_convert_type((ti1 + 127) << 23, jnp.float32)
    pwn_col = jax.lax.bitcast_convert_type((127 - ti1) << 23, jnp.float32)
    cls_small = jax.lax.broadcasted_iota(jnp.int32, (N_OBJS, N_CLASSES), 1)

    def _dot(a, b):
        return jax.lax.dot_general(a, b, (((1,), (0,)), ((), ())),
                                   preferred_element_type=jnp.float32)

    def _exponent(v):
        return (jax.lax.bitcast_convert_type(v, jnp.int32) >> 23) - 127

    for i in range(IMGS):
        t = tgt_ref[i]
        tx0 = t[:, 0:1]
        ty0 = t[:, 1:2]
        tx1 = t[:, 2:3]
        ty1 = t[:, 3:4]
        lab = t[:, 4:5]

        iw = jnp.clip(jnp.minimum(tx1, p_x1) - jnp.maximum(tx0, p_x0),
                      0.0, None)
        ih = jnp.clip(jnp.minimum(ty1, p_y1) - jnp.maximum(ty0, p_y0),
                      0.0, None)
        inter = iw * ih
        area_t = (tx1 - tx0) * (ty1 - ty0)
        ov = inter / (area_t + area_p - inter)

        bto = jnp.max(ov, axis=0, keepdims=True)
        vfirst = _dot(ones_t, jnp.where(ov == bto, pwn_col, 0.0))
        bti = -_exponent(vfirst)

        bpo = jnp.max(ov, axis=1, keepdims=True)
        bpi = jnp.min(jnp.where(ov == bpo, ji, N_PRIORS), axis=1,
                      keepdims=True)

        eq = ji == bpi
        w = _dot(ones_t, jnp.where(eq, pw_col, 0.0))
        forced = w > 0.0
        f_t = _exponent(w)
        bto2 = jnp.where(forced, 2.0, bto)
        bti2 = jnp.where(forced, f_t, bti)

        mself = (ti == bti2).astype(jnp.float32)
        matched = _dot(t.T, mself)
        mx0 = matched[0:1, :]
        my0 = matched[1:2, :]
        mx1 = matched[2:3, :]
        my1 = matched[3:4, :]

        pos = bto2 >= THRESH
        posf = pos.astype(jnp.float32)

        g_cx = ((mx0 + mx1) * 0.5 - p_cx) / (V0 * p_w)
        g_cy = ((my0 + my1) * 0.5 - p_cy) / (V0 * p_h)
        g_w = jnp.log((mx1 - mx0) / p_w) / V1
        g_h = jnp.log((my1 - my0) / p_h) / V1

        l = loc_ref[i]

        def sl1(d):
            a = jnp.abs(d)
            return jnp.where(a < 1.0, 0.5 * d * d, a - 0.5)

        sl = sl1(l[0:1, :] - g_cx) + sl1(l[1:2, :] - g_cy) \
            + sl1(l[2:3, :] - g_w) + sl1(l[3:4, :] - g_h)
        loss_l = jnp.sum(sl * posf)

        c = conf_ref[i].T
        m = jnp.max(c, axis=0, keepdims=True)
        z = jnp.exp(c - m)
        s = _dot(ones_cls, z)
        lse = jnp.log(s) + m

        lcn = jnp.where(pos, 0.0, lse - c[0:1, :])

        lmat = (cls_small == (lab.astype(jnp.int32) + 1)).astype(jnp.float32)
        g = _dot(lmat, c)
        xc_pos = jnp.sum(mself * (posf * g))
        pos_loss = jnp.sum(posf * lse) - xc_pos
        num_pos = jnp.sum(posf)

        lcn_ref[i] = lcn
        stats_ref[i] = jnp.where(lane == 0, loss_l,
                                 jnp.where(lane == 1, pos_loss,
                                           jnp.where(lane == 2, num_pos,
                                                     0.0)))


def _stage2(lcn_ref, stats_ref, out_ref):
    lcn = lcn_ref[...]
    stats = stats_ref[...]
    loss_l = jnp.sum(stats[:, 0, 0:1])
    pos_loss = jnp.sum(stats[:, 0, 1:2])
    npos = stats[:, 0, 2:3]
    n_total = jnp.sum(npos)
    k = jnp.minimum((npos * float(NEGPOS)).astype(jnp.int32),
                    N_PRIORS - 1)
    bits = jax.lax.bitcast_convert_type(lcn, jnp.int32)

    prefix = jnp.zeros((lcn.shape[0], 1), jnp.int32)
    for b in range(30, -1, -1):
        trial = prefix | (1 << b)
        cnt = jnp.sum((bits >= trial).astype(jnp.int32), axis=1,
                      keepdims=True)
        prefix = jnp.where(cnt >= k, trial, prefix)

    cnt_gt = jnp.sum((bits > prefix).astype(jnp.int32), axis=1, keepdims=True)
    sum_gt = jnp.sum(jnp.where(bits > prefix, lcn, 0.0), axis=1,
                     keepdims=True)
    tval = jax.lax.bitcast_convert_type(prefix, jnp.float32)
    neg = sum_gt + (k - cnt_gt).astype(jnp.float32) * tval
    neg_loss = jnp.sum(jnp.where(k > 0, neg, 0.0))

    lane = jax.lax.broadcasted_iota(jnp.int32, (1, 128), 1)
    out_ref[...] = jnp.where(
        lane == 0, loss_l / n_total,
        jnp.where(lane == 1, (pos_loss + neg_loss) / n_total, 0.0))


def kernel(loc_data, conf_data, targets, priors):
    batch = loc_data.shape[0]
    loc_t = jnp.transpose(loc_data, (0, 2, 1))
    pri_t = priors.T

    lcn, stats = pl.pallas_call(
        _stage1,
        grid=(batch // IMGS,),
        in_specs=[
            pl.BlockSpec((IMGS, N_OBJS, 5), lambda b: (b, 0, 0)),
            pl.BlockSpec((4, N_PRIORS), lambda b: (0, 0)),
            pl.BlockSpec((IMGS, 4, N_PRIORS), lambda b: (b, 0, 0)),
            pl.BlockSpec((IMGS, N_PRIORS, N_CLASSES), lambda b: (b, 0, 0)),
        ],
        out_specs=[
            pl.BlockSpec((IMGS, 1, N_PRIORS), lambda b: (b, 0, 0)),
            pl.BlockSpec((IMGS, 1, 128), lambda b: (b, 0, 0)),
        ],
        out_shape=[
            jax.ShapeDtypeStruct((batch, 1, N_PRIORS), jnp.float32),
            jax.ShapeDtypeStruct((batch, 1, 128), jnp.float32),
        ],
    )(targets, pri_t, loc_t, conf_data)

    out = pl.pallas_call(
        _stage2,
        out_shape=jax.ShapeDtypeStruct((1, 128), jnp.float32),
    )(lcn.reshape(batch, N_PRIORS), stats)

    return (out[0, 0], out[0, 1])


# speedup vs baseline: 1.3965x; 1.2440x over previous
import jax
import jax.numpy as jnp
from jax.experimental import pallas as pl

N_CLASSES = 81
THRESH = 0.5
NEGPOS = 3
V0, V1 = 0.1, 0.2
N_PRIORS = 8732
N_OBJS = 20
IMGS = 4


def _stage1(tgt_ref, pri_ref, loc_ref, conf_ref, lcn_ref, stats_ref):
    pri = pri_ref[...]
    p_cx = pri[0:1, :]
    p_cy = pri[1:2, :]
    p_w = pri[2:3, :]
    p_h = pri[3:4, :]
    p_x0 = p_cx - p_w * 0.5
    p_y0 = p_cy - p_h * 0.5
    p_x1 = p_cx + p_w * 0.5
    p_y1 = p_cy + p_h * 0.5
    area_p = (p_x1 - p_x0) * (p_y1 - p_y0)

    ti = jax.lax.broadcasted_iota(jnp.int32, (N_OBJS, N_PRIORS), 0)
    ji = jax.lax.broadcasted_iota(jnp.int32, (N_OBJS, N_PRIORS), 1)
    ones_cls = jnp.ones((1, N_CLASSES), jnp.float32)
    ones_t = jnp.ones((1, N_OBJS), jnp.float32)
    lane = jax.lax.broadcasted_iota(jnp.int32, (1, 128), 1)
    ti1 = jax.lax.broadcasted_iota(jnp.int32, (N_OBJS, 1), 0)
    pw_col = jax.lax.bitcast_convert_type((ti1 + 127) << 23, jnp.float32)
    pwn_col = jax.lax.bitcast_convert_type((127 - ti1) << 23, jnp.float32)
    cls_small = jax.lax.broadcasted_iota(jnp.int32, (N_OBJS, N_CLASSES), 1)

    def _dot(a, b):
        return jax.lax.dot_general(a, b, (((1,), (0,)), ((), ())),
                                   preferred_element_type=jnp.float32)

    def _exponent(v):
        return (jax.lax.bitcast_convert_type(v, jnp.int32) >> 23) - 127

    for i in range(IMGS):
        t = tgt_ref[i]
        tx0 = t[:, 0:1]
        ty0 = t[:, 1:2]
        tx1 = t[:, 2:3]
        ty1 = t[:, 3:4]
        lab = t[:, 4:5]

        iw = jnp.clip(jnp.minimum(tx1, p_x1) - jnp.maximum(tx0, p_x0),
                      0.0, None)
        ih = jnp.clip(jnp.minimum(ty1, p_y1) - jnp.maximum(ty0, p_y0),
                      0.0, None)
        inter = iw * ih
        area_t = (tx1 - tx0) * (ty1 - ty0)
        ov = inter / (area_t + area_p - inter)

        bto = jnp.max(ov, axis=0, keepdims=True)
        vfirst = _dot(ones_t, jnp.where(ov == bto, pwn_col, 0.0))
        bti = -_exponent(vfirst)

        bpo = jnp.max(ov, axis=1, keepdims=True)
        bpi = jnp.min(jnp.where(ov == bpo, ji, N_PRIORS), axis=1,
                      keepdims=True)

        eq = ji == bpi
        w = _dot(ones_t, jnp.where(eq, pw_col, 0.0))
        forced = w > 0.0
        f_t = _exponent(w)
        bto2 = jnp.where(forced, 2.0, bto)
        bti2 = jnp.where(forced, f_t, bti)

        mself = (ti == bti2).astype(jnp.float32)
        matched = _dot(t.T, mself)
        mx0 = matched[0:1, :]
        my0 = matched[1:2, :]
        mx1 = matched[2:3, :]
        my1 = matched[3:4, :]

        pos = bto2 >= THRESH
        posf = pos.astype(jnp.float32)

        g_cx = ((mx0 + mx1) * 0.5 - p_cx) / (V0 * p_w)
        g_cy = ((my0 + my1) * 0.5 - p_cy) / (V0 * p_h)
        g_w = jnp.log((mx1 - mx0) / p_w) / V1
        g_h = jnp.log((my1 - my0) / p_h) / V1

        l = loc_ref[i]

        def sl1(d):
            a = jnp.abs(d)
            return jnp.where(a < 1.0, 0.5 * d * d, a - 0.5)

        sl = sl1(l[0:1, :] - g_cx) + sl1(l[1:2, :] - g_cy) \
            + sl1(l[2:3, :] - g_w) + sl1(l[3:4, :] - g_h)
        loss_l = jnp.sum(sl * posf)

        c = conf_ref[i]
        m = jnp.max(c, axis=0, keepdims=True)
        z = jnp.exp(c - m)
        s = _dot(ones_cls, z)
        lse = jnp.log(s) + m

        lcn = jnp.where(pos, 0.0, lse - c[0:1, :])

        lmat = (cls_small == (lab.astype(jnp.int32) + 1)).astype(jnp.float32)
        g = _dot(lmat, c)
        xc_pos = jnp.sum(mself * (posf * g))
        pos_loss = jnp.sum(posf * lse) - xc_pos
        num_pos = jnp.sum(posf)

        lcn_ref[i] = lcn
        stats_ref[i] = jnp.where(
            lane == 0, loss_l,
            jnp.where(lane == 1, pos_loss,
                      jnp.where(lane == 2, num_pos, 0.0)))


def _stage2(lcn_ref, stats_ref, out_ref):
    lcn = lcn_ref[...]
    stats = stats_ref[...]
    loss_l = jnp.sum(stats[:, 0, 0:1])
    pos_loss = jnp.sum(stats[:, 0, 1:2])
    npos = stats[:, 0, 2:3]
    n_total = jnp.sum(npos)
    k = jnp.minimum((npos * float(NEGPOS)).astype(jnp.int32),
                    N_PRIORS - 1)
    bits = jax.lax.bitcast_convert_type(lcn, jnp.int32)

    prefix = jnp.zeros((lcn.shape[0], 1), jnp.int32)
    for b in range(30, -1, -1):
        trial = prefix | (1 << b)
        cnt = jnp.sum((bits >= trial).astype(jnp.int32), axis=1,
                      keepdims=True)
        prefix = jnp.where(cnt >= k, trial, prefix)

    cnt_gt = jnp.sum((bits > prefix).astype(jnp.int32), axis=1, keepdims=True)
    sum_gt = jnp.sum(jnp.where(bits > prefix, lcn, 0.0), axis=1,
                     keepdims=True)
    tval = jax.lax.bitcast_convert_type(prefix, jnp.float32)
    neg = sum_gt + (k - cnt_gt).astype(jnp.float32) * tval
    neg_loss = jnp.sum(jnp.where(k > 0, neg, 0.0))

    lane = jax.lax.broadcasted_iota(jnp.int32, (1, 128), 1)
    out_ref[...] = jnp.where(
        lane == 0, loss_l / n_total,
        jnp.where(lane == 1, (pos_loss + neg_loss) / n_total, 0.0))


def kernel(loc_data, conf_data, targets, priors):
    batch = loc_data.shape[0]
    pri_t = priors.T
    loc_t = jnp.transpose(loc_data, (0, 2, 1))
    conf_t = jnp.transpose(conf_data, (0, 2, 1))

    lcn, stats = pl.pallas_call(
        _stage1,
        grid=(batch // IMGS,),
        in_specs=[
            pl.BlockSpec((IMGS, N_OBJS, 5), lambda b: (b, 0, 0)),
            pl.BlockSpec((4, N_PRIORS), lambda b: (0, 0)),
            pl.BlockSpec((IMGS, 4, N_PRIORS), lambda b: (b, 0, 0)),
            pl.BlockSpec((IMGS, N_CLASSES, N_PRIORS), lambda b: (b, 0, 0)),
        ],
        out_specs=[
            pl.BlockSpec((IMGS, 1, N_PRIORS), lambda b: (b, 0, 0)),
            pl.BlockSpec((IMGS, 1, 128), lambda b: (b, 0, 0)),
        ],
        out_shape=[
            jax.ShapeDtypeStruct((batch, 1, N_PRIORS), jnp.float32),
            jax.ShapeDtypeStruct((batch, 1, 128), jnp.float32),
        ],
    )(targets, pri_t, loc_t, conf_t)

    out = pl.pallas_call(
        _stage2,
        out_shape=jax.ShapeDtypeStruct((1, 128), jnp.float32),
    )(lcn.reshape(batch, N_PRIORS), stats)

    return (out[0, 0], out[0, 1])
